# baseline (speedup 1.0000x reference)
import jax
import jax.numpy as jnp
from jax import lax
from jax.experimental import pallas as pl
from jax.experimental.pallas import tpu as pltpu
from jax.experimental.pallas import tpu_sc as plsc

F = 26
V = 100000
D = 16
N = 4096
NC = 2
NS = 16
W = NC * NS
C = N // W
CHUNK = 128
NCHUNK = (C * F) // CHUNK


def _body(tab1, tab2, idx2, xv, vmat, cconst, out,
          idx_v, rows_v, first_v, xv_v, v_v, c_v, r_v, out_v, sem2, sem1):
    wid = lax.axis_index("s") * NC + lax.axis_index("c")
    base = wid * C

    pltpu.sync_copy(idx2.at[wid], idx_v)
    pltpu.sync_copy(xv.at[pl.ds(base, C)], xv_v)
    pltpu.sync_copy(vmat, v_v)
    pltpu.sync_copy(cconst, c_v)

    def fire(j, carry):
        pltpu.async_copy(tab2.at[idx_v.at[j]],
                         rows_v.at[pl.ds(j * CHUNK, CHUNK)], sem2)
        pltpu.async_copy(tab1.at[idx_v.at[j]],
                         first_v.at[pl.ds(j * CHUNK, CHUNK)], sem1)
        return carry

    lax.fori_loop(0, NCHUNK, fire, 0)
    pltpu.make_async_copy(tab2.at[pl.ds(0, C * F)], rows_v, sem2).wait()
    pltpu.make_async_copy(tab1.at[pl.ds(0, C * F)], first_v, sem1).wait()

    vrows = [v_v[f] for f in range(F)]
    lanes = lax.iota(jnp.int32, D)
    mask2 = jnp.where(lanes >= 2 * D - F, 1.0, 0.0).astype(jnp.float32)

    def row_body(n, carry):
        rbase = n * F
        xq1 = xv_v[n, pl.ds(0, D)]
        xq2 = xv_v[n, pl.ds(F - D, D)]
        fq1 = first_v[pl.ds(rbase, D)]
        fq2 = first_v[pl.ds(rbase + F - D, D)]
        acc = fq1 * xq1 + mask2 * (fq2 * xq2)
        S = jnp.zeros((D,), jnp.float32)
        A = jnp.zeros((D,), jnp.float32)
        for f in range(F):
            xvs = xq1[f] if f < D else xq2[f - (F - D)]
            e = rows_v[rbase + f] * xvs
            A = A + e * (vrows[f] - 0.5 * e)
            S = S + e
        r_v[pl.ds(n * D, D)] = A + (0.5 * S) * S + acc
        return carry

    lax.fori_loop(0, C, row_body, 0)

    cvec = c_v[...]
    for g in range(C // D):
        oacc = cvec
        ridx = (g * D + lanes) * D
        for d in range(D):
            oacc = oacc + plsc.load_gather(r_v, [ridx + d])
        out_v[pl.ds(g * D, D)] = oacc
    pltpu.sync_copy(out_v, out.at[pl.ds(base, C)])


def kernel(Xi, Xv, first_emb, second_emb, W1, b1, g1, be1, W2, b2, g2, be2,
           bias):
    s = 1.0 / jnp.sqrt(jnp.float32(1.0 + 1e-5))
    u = W2 @ (s * g2)
    vvec = W1 @ (s * g1 * u)
    c = (b1 @ (s * g1 * u) + be1 @ u + b2 @ (s * g2) + jnp.sum(be2)
         + bias[0])

    idx = Xi[:, :, 0].astype(jnp.int32)
    gidx = idx + (jnp.arange(F, dtype=jnp.int32) * V)[None, :]
    idx2 = gidx.reshape(W, NCHUNK, CHUNK)

    tab1 = first_emb.reshape(F * V)
    tab2 = second_emb.reshape(F * V, D)
    vmat = vvec.reshape(F, D)
    cconst = jnp.full((D,), c, jnp.float32)

    mesh = plsc.VectorSubcoreMesh(core_axis_name="c", subcore_axis_name="s",
                                  num_cores=NC, num_subcores=NS)
    run = pl.kernel(
        _body,
        out_type=jax.ShapeDtypeStruct((N,), jnp.float32),
        mesh=mesh,
        compiler_params=pltpu.CompilerParams(needs_layout_passes=False,
                                             use_tc_tiling_on_sc=False),
        scratch_types=[
            pltpu.VMEM((NCHUNK, CHUNK), jnp.int32),
            pltpu.VMEM((C * F, D), jnp.float32),
            pltpu.VMEM((C * F,), jnp.float32),
            pltpu.VMEM((C, F), jnp.float32),
            pltpu.VMEM((F, D), jnp.float32),
            pltpu.VMEM((D,), jnp.float32),
            pltpu.VMEM((C * D,), jnp.float32),
            pltpu.VMEM((C,), jnp.float32),
            pltpu.SemaphoreType.DMA,
            pltpu.SemaphoreType.DMA,
        ],
    )
    return run(tab1, tab2, idx2, Xv, vmat, cconst)


# speedup vs baseline: 2.2771x; 2.2771x over previous
import jax
import jax.numpy as jnp
from jax import lax
from jax.experimental import pallas as pl
from jax.experimental.pallas import tpu as pltpu
from jax.experimental.pallas import tpu_sc as plsc

F = 26
V = 100000
D = 16
N = 4096
NC = 2
NS = 16
W = NC * NS
C = N // W
CHUNK = 128
NCHUNK = (C * F) // CHUNK
NG = (C * F * D) // CHUNK


def _body(tab1, tab2, idx2, idx1, xv, vmat, cconst, out,
          bas_v, idxg, vals, idx_v, first_v, xv_v, v_v, c_v, r_v, out_v,
          sem2, sem3, sem1):
    wid = lax.axis_index("s") * NC + lax.axis_index("c")
    base = wid * C

    pltpu.sync_copy(idx2.at[wid], bas_v)
    pltpu.sync_copy(idx1.at[wid], idx_v)
    pltpu.sync_copy(xv.at[pl.ds(base, C)], xv_v)
    pltpu.sync_copy(vmat, v_v)
    pltpu.sync_copy(cconst, c_v)

    def fire1(j, carry):
        pltpu.async_copy(tab1.at[idx_v.at[j]],
                         first_v.at[pl.ds(j * CHUNK, CHUNK)], sem1)
        return carry

    lax.fori_loop(0, NCHUNK, fire1, 0)

    lanes = lax.iota(jnp.int32, D)
    lanesV = lanes * V

    def make_expand(sem):
        def expand(j, carry):
            for b in range(8):
                bv = bas_v[j, pl.ds(b * D, D)]
                for l in range(D):
                    q = b * D + l
                    idxg[j * D + q // 8, pl.ds((q % 8) * D, D)] = (
                        bv[l] + lanesV)
            for t in range(D):
                pltpu.async_copy(tab2.at[idxg.at[j * D + t]],
                                 vals.at[pl.ds((j * D + t) * CHUNK, CHUNK)],
                                 sem)
            return carry
        return expand

    HALFW = C * F * D // 2
    lax.fori_loop(0, NCHUNK // 2, make_expand(sem2), 0)
    lax.fori_loop(NCHUNK // 2, NCHUNK, make_expand(sem3), 0)

    pltpu.make_async_copy(tab1.at[pl.ds(0, C * F)], first_v, sem1).wait()
    pltpu.make_async_copy(tab2.at[pl.ds(0, HALFW)],
                          vals.at[pl.ds(0, HALFW)], sem2).wait()

    vrows = [v_v[f] for f in range(F)]
    mask2 = jnp.where(lanes >= 2 * D - F, 1.0, 0.0).astype(jnp.float32)

    def row_body(n, carry):
        rbase = n * F
        xq1 = xv_v[n, pl.ds(0, D)]
        xq2 = xv_v[n, pl.ds(F - D, D)]
        fq1 = first_v[pl.ds(rbase, D)]
        fq2 = first_v[pl.ds(rbase + F - D, D)]
        acc = fq1 * xq1 + mask2 * (fq2 * xq2)
        S = jnp.zeros((D,), jnp.float32)
        A = jnp.zeros((D,), jnp.float32)
        for f in range(F):
            xvs = xq1[f] if f < D else xq2[f - (F - D)]
            e = vals[pl.ds((rbase + f) * D, D)] * xvs
            A = A + e * (vrows[f] - 0.5 * e)
            S = S + e
        r_v[pl.ds(n * D, D)] = A + (0.5 * S) * S + acc
        return carry

    lax.fori_loop(0, C // 2, row_body, 0)
    pltpu.make_async_copy(tab2.at[pl.ds(0, HALFW)],
                          vals.at[pl.ds(HALFW, HALFW)], sem3).wait()
    lax.fori_loop(C // 2, C, row_body, 0)

    cvec = c_v[...]
    for g in range(C // D):
        oacc = cvec
        ridx = (g * D + lanes) * D
        for d in range(D):
            oacc = oacc + plsc.load_gather(r_v, [ridx + d])
        out_v[pl.ds(g * D, D)] = oacc
    pltpu.sync_copy(out_v, out.at[pl.ds(base, C)])


def kernel(Xi, Xv, first_emb, second_emb, W1, b1, g1, be1, W2, b2, g2, be2,
           bias):
    s = 1.0 / jnp.sqrt(jnp.float32(1.0 + 1e-5))
    u = W2 @ (s * g2)
    vvec = W1 @ (s * g1 * u)
    c = (b1 @ (s * g1 * u) + be1 @ u + b2 @ (s * g2) + jnp.sum(be2)
         + bias[0])

    idx = Xi[:, :, 0].astype(jnp.int32)
    fr = jnp.arange(F, dtype=jnp.int32)[None, :]
    idx2 = (idx + fr * (D * V)).reshape(W, NCHUNK, CHUNK)
    idx1 = (idx + fr * V).reshape(W, NCHUNK, CHUNK)

    tab1 = first_emb.reshape(F * V)
    tab2 = jnp.transpose(second_emb, (0, 2, 1)).reshape(F * D * V)
    vmat = vvec.reshape(F, D)
    cconst = jnp.full((D,), c, jnp.float32)

    mesh = plsc.VectorSubcoreMesh(core_axis_name="c", subcore_axis_name="s",
                                  num_cores=NC, num_subcores=NS)
    run = pl.kernel(
        _body,
        out_type=jax.ShapeDtypeStruct((N,), jnp.float32),
        mesh=mesh,
        compiler_params=pltpu.CompilerParams(needs_layout_passes=False,
                                             use_tc_tiling_on_sc=False),
        scratch_types=[
            pltpu.VMEM((NCHUNK, CHUNK), jnp.int32),
            pltpu.VMEM((NG, CHUNK), jnp.int32),
            pltpu.VMEM((C * F * D,), jnp.float32),
            pltpu.VMEM((NCHUNK, CHUNK), jnp.int32),
            pltpu.VMEM((C * F,), jnp.float32),
            pltpu.VMEM((C, F), jnp.float32),
            pltpu.VMEM((F, D), jnp.float32),
            pltpu.VMEM((D,), jnp.float32),
            pltpu.VMEM((C * D,), jnp.float32),
            pltpu.VMEM((C,), jnp.float32),
            pltpu.SemaphoreType.DMA,
            pltpu.SemaphoreType.DMA,
            pltpu.SemaphoreType.DMA,
        ],
    )
    return run(tab1, tab2, idx2, idx1, Xv, vmat, cconst)
